# baseline (speedup 1.0000x reference)
import jax
import jax.numpy as jnp
import numpy as np
from jax import lax
from jax.experimental import pallas as pl
from jax.experimental.pallas import tpu as pltpu
from jax.experimental.pallas import tpu_sc as plsc

D = 64
L = 200
CB = 4
NV = D // 16


OB = 8
RB = 128


def _transpose_body(tt_hbm, scr_hbm, tin, tin_tail, tout):
    nc = 2
    ns = 16
    wid = lax.axis_index("c") * ns + lax.axis_index("s")
    nw = nc * ns
    v = scr_hbm.shape[0]
    nblk = v // RB
    tail0 = nblk * RB
    tail_n = v - tail0
    per = nblk // nw
    rem = nblk % nw
    lo = wid * per + jnp.minimum(wid, rem)
    cnt = per + jnp.where(wid < rem, 1, 0)
    iota = lax.iota(jnp.int32, 16)

    def transpose_rows(tin_ref, nrows):
        @pl.loop(0, nrows)
        def _row(r):
            ridx = jnp.broadcast_to(r, (16,)).astype(jnp.int32)
            for k in range(D // 16):
                vvec = plsc.load_gather(tin_ref, [16 * k + iota, ridx])
                tout[r, pl.ds(16 * k, 16)] = vvec

    @pl.loop(0, cnt)
    def _blk(t):
        col0 = pl.multiple_of((lo + t) * RB, RB)
        pltpu.sync_copy(tt_hbm.at[:, pl.ds(col0, RB)], tin)
        transpose_rows(tin, RB)
        pltpu.sync_copy(tout, scr_hbm.at[pl.ds(col0, RB), :])

    if tail_n:
        @pl.when(wid == nw - 1)
        def _tail():
            pltpu.sync_copy(tt_hbm.at[:, pl.ds(tail0, tail_n)], tin_tail)
            transpose_rows(tin_tail, tail_n)
            pltpu.sync_copy(
                tout.at[pl.ds(0, tail_n), :], scr_hbm.at[pl.ds(tail0, tail_n), :]
            )


def _cbow_body(ids_hbm, table_hbm, out_hbm,
               idx0, idx1, rows0, rows1, ostag, sem0, sem1):
    nc = 2
    ns = 16
    wid = lax.axis_index("c") * ns + lax.axis_index("s")
    b_total = out_hbm.shape[0]
    rows_per_w = b_total // (nc * ns)
    n_chunks = rows_per_w // CB
    row0 = wid * rows_per_w
    inv_l = np.float32(1.0 / L)
    bufs = ((idx0, rows0, sem0), (idx1, rows1, sem1))

    def load_idx_and_fire(g, idxb, rowsb, semb):
        r0 = row0 + g * CB
        pltpu.sync_copy(ids_hbm.at[pl.ds(r0 * L, CB * L)], idxb)
        pltpu.async_copy(table_hbm.at[idxb], rowsb, semb)

    load_idx_and_fire(0, *bufs[0])

    @pl.loop(0, n_chunks, step=2)
    def _chunk2(g2):
        for b in (0, 1):
            g = g2 + b
            idxb, rowsb, semb = bufs[b]

            @pl.when(g + 1 < n_chunks)
            def _fire_next():
                load_idx_and_fire(g + 1, *bufs[1 - b])

            pltpu.make_async_copy(table_hbm.at[idxb], rowsb, semb).wait()

            pos = g % OB
            for cb in range(CB):
                zero = jnp.zeros((16,), jnp.float32)

                @pl.loop(0, L, init_carry=(zero,) * NV, unroll=8)
                def _acc(l, carry):
                    r = cb * L + l
                    return tuple(
                        carry[d] + rowsb[r, pl.ds(d * 16, 16)]
                        for d in range(NV)
                    )

                for d in range(NV):
                    ostag[pos * CB + cb, pl.ds(d * 16, 16)] = _acc[d] * inv_l

            @pl.when(pos == OB - 1)
            def _flush_out():
                pltpu.sync_copy(
                    ostag,
                    out_hbm.at[pl.ds(row0 + (g + 1 - OB) * CB, OB * CB)],
                )


@jax.jit
def kernel(input_ids, table):
    b, l = input_ids.shape
    v = table.shape[0]
    assert l == L and table.shape[1] == D

    mesh = plsc.VectorSubcoreMesh(core_axis_name="c", subcore_axis_name="s")

    k1 = pl.kernel(
        _transpose_body,
        out_type=jax.ShapeDtypeStruct((v, 2 * D), jnp.float32),
        mesh=mesh,
        scratch_types=[
            pltpu.VMEM((D, RB), jnp.float32),
            pltpu.VMEM((D, D), jnp.float32),
            pltpu.VMEM((RB, 2 * D), jnp.float32),
        ],
        compiler_params=pltpu.CompilerParams(needs_layout_passes=False),
    )
    scratch = k1(table.T)

    table2 = scratch.reshape(2 * v, D)
    ids_flat = input_ids.reshape(-1).astype(jnp.int32) * 2

    k2 = pl.kernel(
        _cbow_body,
        out_type=jax.ShapeDtypeStruct((b, D), jnp.float32),
        mesh=mesh,
        scratch_types=[
            pltpu.VMEM((CB * L,), jnp.int32),
            pltpu.VMEM((CB * L,), jnp.int32),
            pltpu.VMEM((CB * L, D), jnp.float32),
            pltpu.VMEM((CB * L, D), jnp.float32),
            pltpu.VMEM((OB * CB, D), jnp.float32),
            pltpu.SemaphoreType.DMA,
            pltpu.SemaphoreType.DMA,
        ],
        compiler_params=pltpu.CompilerParams(use_tc_tiling_on_sc=False),
    )
    return k2(ids_flat, table2)


# speedup vs baseline: 1.4506x; 1.4506x over previous
import jax
import jax.numpy as jnp
import numpy as np
from jax import lax
from jax.experimental import pallas as pl
from jax.experimental.pallas import tpu as pltpu
from jax.experimental.pallas import tpu_sc as plsc

D = 64
L = 200
CB = 4
NV = D // 16


OB = 8
RB = 256


def _transpose_body(tt_hbm, scr_hbm,
                    tin0, tin1, tout0, tout1, tin_tail,
                    sin0, sin1, sout0, sout1):
    nc = 2
    ns = 16
    wid = lax.axis_index("c") * ns + lax.axis_index("s")
    nw = nc * ns
    v = scr_hbm.shape[0] // D
    nblk = v // RB
    tail0 = nblk * RB
    tail_n = v - tail0
    per = nblk // nw
    rem = nblk % nw
    lo = wid * per + jnp.minimum(wid, rem)
    cnt = per + jnp.where(wid < rem, 1, 0)
    iota_d = lax.iota(jnp.int32, 16) * D
    bufs = ((tin0, tout0, sin0, sout0), (tin1, tout1, sin1, sout1))

    def col_of(t):
        return pl.multiple_of((lo + t) * RB, RB)

    def fire_in(t, tin, sin):
        pltpu.async_copy(tt_hbm.at[:, pl.ds(col_of(t), RB)], tin, sin)

    def transpose_buf(tin, tout, nrows):
        @pl.loop(0, nrows, step=16)
        def _r(r0):
            base = r0 * D
            for c in range(D):
                plsc.store_scatter(
                    tout, [iota_d + (base + c)], tin[c, pl.ds(r0, 16)]
                )

    fire_in(0, tin0, sin0)

    @pl.loop(0, per + 1, step=2)
    def _pair(i2):
        for b in (0, 1):
            t = i2 + b
            tin, tout, sin, sout = bufs[b]
            tin_n, _, sin_n, _ = bufs[1 - b]

            @pl.when(t < cnt)
            def _do():
                @pl.when(t + 1 < cnt)
                def _fire_next():
                    fire_in(t + 1, tin_n, sin_n)

                pltpu.make_async_copy(
                    tt_hbm.at[:, pl.ds(0, RB)], tin, sin
                ).wait()

                @pl.when(t >= 2)
                def _drain_prev_out():
                    pltpu.make_async_copy(
                        tout, scr_hbm.at[pl.ds(0, RB * D)], sout
                    ).wait()

                transpose_buf(tin, tout, RB)
                pltpu.async_copy(
                    tout, scr_hbm.at[pl.ds(col_of(t) * D, RB * D)], sout
                )

    for _, tout, _, sout in bufs:
        pltpu.make_async_copy(tout, scr_hbm.at[pl.ds(0, RB * D)], sout).wait()

    if tail_n:
        @pl.when(wid == nw - 1)
        def _tail():
            pltpu.sync_copy(tt_hbm.at[:, pl.ds(tail0, tail_n)], tin_tail)

            @pl.loop(0, tail_n, step=16)
            def _r(r0):
                base = r0 * D
                for c in range(D):
                    plsc.store_scatter(
                        tout0, [iota_d + (base + c)], tin_tail[c, pl.ds(r0, 16)]
                    )

            pltpu.sync_copy(
                tout0.at[pl.ds(0, tail_n * D)],
                scr_hbm.at[pl.ds(tail0 * D, tail_n * D)],
            )


def _cbow_body(ids_hbm, table_hbm, out_hbm,
               idx0, idx1, rows0, rows1, ostag, sem0, sem1):
    nc = 2
    ns = 16
    wid = lax.axis_index("c") * ns + lax.axis_index("s")
    b_total = out_hbm.shape[0]
    rows_per_w = b_total // (nc * ns)
    n_chunks = rows_per_w // CB
    row0 = wid * rows_per_w
    inv_l = np.float32(1.0 / L)
    bufs = ((idx0, rows0, sem0), (idx1, rows1, sem1))

    def load_idx_and_fire(g, idxb, rowsb, semb):
        r0 = row0 + g * CB
        pltpu.sync_copy(ids_hbm.at[pl.ds(r0 * L, CB * L)], idxb)
        pltpu.async_copy(table_hbm.at[idxb], rowsb, semb)

    load_idx_and_fire(0, *bufs[0])

    @pl.loop(0, n_chunks, step=2)
    def _chunk2(g2):
        for b in (0, 1):
            g = g2 + b
            idxb, rowsb, semb = bufs[b]

            @pl.when(g + 1 < n_chunks)
            def _fire_next():
                load_idx_and_fire(g + 1, *bufs[1 - b])

            pltpu.make_async_copy(table_hbm.at[idxb], rowsb, semb).wait()

            pos = g % OB
            for cb in range(CB):
                zero = jnp.zeros((16,), jnp.float32)

                @pl.loop(0, L, init_carry=(zero,) * NV, unroll=8)
                def _acc(l, carry):
                    r = cb * L + l
                    return tuple(
                        carry[d] + rowsb[r, pl.ds(d * 16, 16)]
                        for d in range(NV)
                    )

                for d in range(NV):
                    ostag[pos * CB + cb, pl.ds(d * 16, 16)] = _acc[d] * inv_l

            @pl.when(pos == OB - 1)
            def _flush_out():
                pltpu.sync_copy(
                    ostag,
                    out_hbm.at[pl.ds(row0 + (g + 1 - OB) * CB, OB * CB)],
                )


@jax.jit
def kernel(input_ids, table):
    b, l = input_ids.shape
    v = table.shape[0]
    assert l == L and table.shape[1] == D

    mesh = plsc.VectorSubcoreMesh(core_axis_name="c", subcore_axis_name="s")

    k1 = pl.kernel(
        _transpose_body,
        out_type=jax.ShapeDtypeStruct((v * D,), jnp.float32),
        mesh=mesh,
        scratch_types=[
            pltpu.VMEM((D, RB), jnp.float32),
            pltpu.VMEM((D, RB), jnp.float32),
            pltpu.VMEM((RB * D,), jnp.float32),
            pltpu.VMEM((RB * D,), jnp.float32),
            pltpu.VMEM((D, D), jnp.float32),
            pltpu.SemaphoreType.DMA,
            pltpu.SemaphoreType.DMA,
            pltpu.SemaphoreType.DMA,
            pltpu.SemaphoreType.DMA,
        ],
        compiler_params=pltpu.CompilerParams(needs_layout_passes=False),
    )
    scratch = k1(table.T)

    table2 = scratch.reshape(v, D)
    ids_flat = input_ids.reshape(-1).astype(jnp.int32)

    k2 = pl.kernel(
        _cbow_body,
        out_type=jax.ShapeDtypeStruct((b, D), jnp.float32),
        mesh=mesh,
        scratch_types=[
            pltpu.VMEM((CB * L,), jnp.int32),
            pltpu.VMEM((CB * L,), jnp.int32),
            pltpu.VMEM((CB * L, D), jnp.float32),
            pltpu.VMEM((CB * L, D), jnp.float32),
            pltpu.VMEM((OB * CB, D), jnp.float32),
            pltpu.SemaphoreType.DMA,
            pltpu.SemaphoreType.DMA,
        ],
        compiler_params=pltpu.CompilerParams(use_tc_tiling_on_sc=False),
    )
    return k2(ids_flat, table2)


# speedup vs baseline: 1.8515x; 1.2763x over previous
import jax
import jax.numpy as jnp
import numpy as np
from jax import lax
from jax.experimental import pallas as pl
from jax.experimental.pallas import tpu as pltpu
from jax.experimental.pallas import tpu_sc as plsc

D = 64
L = 200
CB = 4
NV = D // 16


OB = 8
RB = 256


def _transpose_body(tt_hbm, scr_hbm,
                    tin0, tin1, tout0, tout1, tin_tail, skew,
                    sin0, sin1, sout0, sout1):
    nc = 2
    ns = 16
    wid = lax.axis_index("c") * ns + lax.axis_index("s")
    nw = nc * ns
    v = scr_hbm.shape[0] // D
    nblk = v // RB
    tail0 = nblk * RB
    tail_n = v - tail0
    per = nblk // nw
    rem = nblk % nw
    lo = wid * per + jnp.minimum(wid, rem)
    cnt = per + jnp.where(wid < rem, 1, 0)
    iota_sk = lax.iota(jnp.int32, 16) * (D + 1)
    bufs = ((tin0, tout0, sin0, sout0), (tin1, tout1, sin1, sout1))

    def col_of(t):
        return pl.multiple_of((lo + t) * RB, RB)

    def fire_in(t, tin, sin):
        pltpu.async_copy(tt_hbm.at[:, pl.ds(col_of(t), RB)], tin, sin)

    def transpose_buf(tin, tout, skew, nrows):
        @pl.loop(0, nrows, step=16)
        def _r(r0):
            for c in range(D):
                plsc.store_scatter(skew, [iota_sk + c], tin[c, pl.ds(r0, 16)])
            for k in range(16):
                obase = (r0 + k) * D
                sbase = k * (D + 1)
                for q in range(D // 16):
                    tout[pl.ds(obase + q * 16, 16)] = skew[pl.ds(sbase + q * 16, 16)]

    fire_in(0, tin0, sin0)

    @pl.loop(0, per + 1, step=2)
    def _pair(i2):
        for b in (0, 1):
            t = i2 + b
            tin, tout, sin, sout = bufs[b]
            tin_n, _, sin_n, _ = bufs[1 - b]

            @pl.when(t < cnt)
            def _do():
                @pl.when(t + 1 < cnt)
                def _fire_next():
                    fire_in(t + 1, tin_n, sin_n)

                pltpu.make_async_copy(
                    tt_hbm.at[:, pl.ds(0, RB)], tin, sin
                ).wait()

                @pl.when(t >= 2)
                def _drain_prev_out():
                    pltpu.make_async_copy(
                        tout, scr_hbm.at[pl.ds(0, RB * D)], sout
                    ).wait()

                transpose_buf(tin, tout, skew, RB)
                pltpu.async_copy(
                    tout, scr_hbm.at[pl.ds(col_of(t) * D, RB * D)], sout
                )

    for _, tout, _, sout in bufs:
        pltpu.make_async_copy(tout, scr_hbm.at[pl.ds(0, RB * D)], sout).wait()

    if tail_n:
        @pl.when(wid == nw - 1)
        def _tail():
            pltpu.sync_copy(tt_hbm.at[:, pl.ds(tail0, tail_n)], tin_tail)
            transpose_buf(tin_tail, tout0, skew, tail_n)
            pltpu.sync_copy(
                tout0.at[pl.ds(0, tail_n * D)],
                scr_hbm.at[pl.ds(tail0 * D, tail_n * D)],
            )


def _cbow_body(ids_hbm, table_hbm, out_hbm,
               idx0, idx1, rows0, rows1, ostag, sem0, sem1):
    nc = 2
    ns = 16
    wid = lax.axis_index("c") * ns + lax.axis_index("s")
    b_total = out_hbm.shape[0]
    rows_per_w = b_total // (nc * ns)
    n_chunks = rows_per_w // CB
    row0 = wid * rows_per_w
    inv_l = np.float32(1.0 / L)
    bufs = ((idx0, rows0, sem0), (idx1, rows1, sem1))

    def load_idx_and_fire(g, idxb, rowsb, semb):
        r0 = row0 + g * CB
        pltpu.sync_copy(ids_hbm.at[pl.ds(r0 * L, CB * L)], idxb)
        pltpu.async_copy(table_hbm.at[idxb], rowsb, semb)

    load_idx_and_fire(0, *bufs[0])

    @pl.loop(0, n_chunks, step=2)
    def _chunk2(g2):
        for b in (0, 1):
            g = g2 + b
            idxb, rowsb, semb = bufs[b]

            @pl.when(g + 1 < n_chunks)
            def _fire_next():
                load_idx_and_fire(g + 1, *bufs[1 - b])

            pltpu.make_async_copy(table_hbm.at[idxb], rowsb, semb).wait()

            pos = g % OB
            for cb in range(CB):
                zero = jnp.zeros((16,), jnp.float32)

                @pl.loop(0, L, init_carry=(zero,) * NV, unroll=8)
                def _acc(l, carry):
                    r = cb * L + l
                    return tuple(
                        carry[d] + rowsb[r, pl.ds(d * 16, 16)]
                        for d in range(NV)
                    )

                for d in range(NV):
                    ostag[pos * CB + cb, pl.ds(d * 16, 16)] = _acc[d] * inv_l

            @pl.when(pos == OB - 1)
            def _flush_out():
                pltpu.sync_copy(
                    ostag,
                    out_hbm.at[pl.ds(row0 + (g + 1 - OB) * CB, OB * CB)],
                )


@jax.jit
def kernel(input_ids, table):
    b, l = input_ids.shape
    v = table.shape[0]
    assert l == L and table.shape[1] == D

    mesh = plsc.VectorSubcoreMesh(core_axis_name="c", subcore_axis_name="s")

    k1 = pl.kernel(
        _transpose_body,
        out_type=jax.ShapeDtypeStruct((v * D,), jnp.float32),
        mesh=mesh,
        scratch_types=[
            pltpu.VMEM((D, RB), jnp.float32),
            pltpu.VMEM((D, RB), jnp.float32),
            pltpu.VMEM((RB * D,), jnp.float32),
            pltpu.VMEM((RB * D,), jnp.float32),
            pltpu.VMEM((D, D), jnp.float32),
            pltpu.VMEM((16 * (D + 1),), jnp.float32),
            pltpu.SemaphoreType.DMA,
            pltpu.SemaphoreType.DMA,
            pltpu.SemaphoreType.DMA,
            pltpu.SemaphoreType.DMA,
        ],
        compiler_params=pltpu.CompilerParams(needs_layout_passes=False),
    )
    scratch = k1(table.T)

    table2 = scratch.reshape(v, D)
    ids_flat = input_ids.reshape(-1).astype(jnp.int32)

    k2 = pl.kernel(
        _cbow_body,
        out_type=jax.ShapeDtypeStruct((b, D), jnp.float32),
        mesh=mesh,
        scratch_types=[
            pltpu.VMEM((CB * L,), jnp.int32),
            pltpu.VMEM((CB * L,), jnp.int32),
            pltpu.VMEM((CB * L, D), jnp.float32),
            pltpu.VMEM((CB * L, D), jnp.float32),
            pltpu.VMEM((OB * CB, D), jnp.float32),
            pltpu.SemaphoreType.DMA,
            pltpu.SemaphoreType.DMA,
        ],
        compiler_params=pltpu.CompilerParams(use_tc_tiling_on_sc=False),
    )
    return k2(ids_flat, table2)


# speedup vs baseline: 1.8522x; 1.0004x over previous
import jax
import jax.numpy as jnp
import numpy as np
from jax import lax
from jax.experimental import pallas as pl
from jax.experimental.pallas import tpu as pltpu
from jax.experimental.pallas import tpu_sc as plsc

D = 64
L = 200
CB = 4
NV = D // 16


OB = 8
RB = 256


def _transpose_body(tt_hbm, scr_hbm,
                    tin0, tin1, tout0, tout1, tin_tail, skew,
                    sin0, sin1, sout0, sout1):
    nc = 2
    ns = 16
    wid = lax.axis_index("c") * ns + lax.axis_index("s")
    nw = nc * ns
    v = scr_hbm.shape[0] // D
    nblk = v // RB
    tail0 = nblk * RB
    tail_n = v - tail0
    per = nblk // nw
    rem = nblk % nw
    lo = wid * per + jnp.minimum(wid, rem)
    cnt = per + jnp.where(wid < rem, 1, 0)
    iota_sk = lax.iota(jnp.int32, 16) * (D + 1)
    bufs = ((tin0, tout0, sin0, sout0), (tin1, tout1, sin1, sout1))

    def col_of(t):
        return pl.multiple_of((lo + t) * RB, RB)

    def fire_in(t, tin, sin):
        pltpu.async_copy(tt_hbm.at[:, pl.ds(col_of(t), RB)], tin, sin)

    def transpose_buf(tin, tout, skew, nrows):
        @pl.loop(0, nrows, step=16)
        def _r(r0):
            for c in range(D):
                plsc.store_scatter(skew, [iota_sk + c], tin[c, pl.ds(r0, 16)])
            for k in range(16):
                obase = (r0 + k) * D
                sbase = k * (D + 1)
                for q in range(D // 16):
                    tout[pl.ds(obase + q * 16, 16)] = skew[pl.ds(sbase + q * 16, 16)]

    fire_in(0, tin0, sin0)

    @pl.loop(0, per + 1, step=2)
    def _pair(i2):
        for b in (0, 1):
            t = i2 + b
            tin, tout, sin, sout = bufs[b]
            tin_n, _, sin_n, _ = bufs[1 - b]

            @pl.when(t < cnt)
            def _do():
                @pl.when(t + 1 < cnt)
                def _fire_next():
                    fire_in(t + 1, tin_n, sin_n)

                pltpu.make_async_copy(
                    tt_hbm.at[:, pl.ds(0, RB)], tin, sin
                ).wait()

                @pl.when(t >= 2)
                def _drain_prev_out():
                    pltpu.make_async_copy(
                        tout, scr_hbm.at[pl.ds(0, RB * D)], sout
                    ).wait()

                transpose_buf(tin, tout, skew, RB)
                pltpu.async_copy(
                    tout, scr_hbm.at[pl.ds(col_of(t) * D, RB * D)], sout
                )

    for _, tout, _, sout in bufs:
        pltpu.make_async_copy(tout, scr_hbm.at[pl.ds(0, RB * D)], sout).wait()

    if tail_n:
        @pl.when(wid == nw - 1)
        def _tail():
            pltpu.sync_copy(tt_hbm.at[:, pl.ds(tail0, tail_n)], tin_tail)
            transpose_buf(tin_tail, tout0, skew, tail_n)
            pltpu.sync_copy(
                tout0.at[pl.ds(0, tail_n * D)],
                scr_hbm.at[pl.ds(tail0 * D, tail_n * D)],
            )


def _cbow_body(ids_hbm, table_hbm, out_hbm,
               idx0, idx1, rows0, rows1, ostag, sem0, sem1):
    nc = 2
    ns = 16
    wid = lax.axis_index("c") * ns + lax.axis_index("s")
    b_total = out_hbm.shape[0]
    rows_per_w = b_total // (nc * ns)
    n_chunks = rows_per_w // CB
    row0 = wid * rows_per_w
    inv_l = np.float32(1.0 / L)
    bufs = ((idx0, rows0, sem0), (idx1, rows1, sem1))

    def load_idx_and_fire(g, idxb, rowsb, semb):
        r0 = row0 + g * CB
        pltpu.sync_copy(ids_hbm.at[pl.ds(r0 * L, CB * L)], idxb)
        pltpu.async_copy(table_hbm.at[idxb], rowsb, semb)

    load_idx_and_fire(0, *bufs[0])

    @pl.loop(0, n_chunks, step=2)
    def _chunk2(g2):
        for b in (0, 1):
            g = g2 + b
            idxb, rowsb, semb = bufs[b]

            @pl.when(g + 1 < n_chunks)
            def _fire_next():
                load_idx_and_fire(g + 1, *bufs[1 - b])

            pltpu.make_async_copy(table_hbm.at[idxb], rowsb, semb).wait()

            pos = g % OB
            for cb in range(CB):
                zero = jnp.zeros((16,), jnp.float32)

                @pl.loop(0, L, init_carry=(zero,) * NV, unroll=8)
                def _acc(l, carry):
                    r = cb * L + l
                    return tuple(
                        carry[d] + rowsb[r, pl.ds(d * 16, 16)]
                        for d in range(NV)
                    )

                for d in range(NV):
                    ostag[pos * CB + cb, pl.ds(d * 16, 16)] = _acc[d] * inv_l

            @pl.when(pos == OB - 1)
            def _flush_out():
                pltpu.sync_copy(
                    ostag,
                    out_hbm.at[pl.ds(row0 + (g + 1 - OB) * CB, OB * CB)],
                )


@jax.jit
def kernel(input_ids, table):
    b, l = input_ids.shape
    v = table.shape[0]
    assert l == L and table.shape[1] == D

    mesh = plsc.VectorSubcoreMesh(core_axis_name="c", subcore_axis_name="s")

    k1 = pl.kernel(
        _transpose_body,
        out_type=jax.ShapeDtypeStruct((v * D,), jnp.float32),
        mesh=mesh,
        scratch_types=[
            pltpu.VMEM((D, RB), jnp.float32),
            pltpu.VMEM((D, RB), jnp.float32),
            pltpu.VMEM((RB * D,), jnp.float32),
            pltpu.VMEM((RB * D,), jnp.float32),
            pltpu.VMEM((D, D), jnp.float32),
            pltpu.VMEM((16 * (D + 1),), jnp.float32),
            pltpu.SemaphoreType.DMA,
            pltpu.SemaphoreType.DMA,
            pltpu.SemaphoreType.DMA,
            pltpu.SemaphoreType.DMA,
        ],
        compiler_params=pltpu.CompilerParams(
            needs_layout_passes=False, disable_bounds_checks=True
        ),
    )
    scratch = k1(table.T)

    table2 = scratch.reshape(v, D)
    ids_flat = input_ids.reshape(-1).astype(jnp.int32)

    k2 = pl.kernel(
        _cbow_body,
        out_type=jax.ShapeDtypeStruct((b, D), jnp.float32),
        mesh=mesh,
        scratch_types=[
            pltpu.VMEM((CB * L,), jnp.int32),
            pltpu.VMEM((CB * L,), jnp.int32),
            pltpu.VMEM((CB * L, D), jnp.float32),
            pltpu.VMEM((CB * L, D), jnp.float32),
            pltpu.VMEM((OB * CB, D), jnp.float32),
            pltpu.SemaphoreType.DMA,
            pltpu.SemaphoreType.DMA,
        ],
        compiler_params=pltpu.CompilerParams(use_tc_tiling_on_sc=False),
    )
    return k2(ids_flat, table2)


# speedup vs baseline: 2.1173x; 1.1431x over previous
import jax
import jax.numpy as jnp
import numpy as np
from jax import lax
from jax.experimental import pallas as pl
from jax.experimental.pallas import tpu as pltpu
from jax.experimental.pallas import tpu_sc as plsc

D = 64
L = 200
CB = 4
NV = D // 16


OB = 8
RB = 256


def _transpose_body(tt_hbm, scr_hbm,
                    tin0, tin1, tout0, tout1, tin_tail, skew,
                    sin0, sin1, sout0, sout1):
    nc = 2
    ns = 16
    wid = lax.axis_index("c") * ns + lax.axis_index("s")
    nw = nc * ns
    v = scr_hbm.shape[0] // D
    nblk = v // RB
    tail0 = nblk * RB
    tail_n = v - tail0
    per = nblk // nw
    rem = nblk % nw
    lo = wid * per + jnp.minimum(wid, rem)
    cnt = per + jnp.where(wid < rem, 1, 0)
    iota_sk = lax.iota(jnp.int32, 16) * (D + 1)
    bufs = ((tin0, tout0, sin0, sout0), (tin1, tout1, sin1, sout1))

    def col_of(t):
        return pl.multiple_of((lo + t) * RB, RB)

    def fire_in(t, tin, sin):
        pltpu.async_copy(tt_hbm.at[:, pl.ds(col_of(t), RB)], tin, sin)

    skslot = 16 * (D + 1)

    def transpose_buf(tin, tout, skew, nrows):
        @plsc.parallel_loop(0, nrows // 16, 1, unroll=2)
        def _grp(i):
            r0 = i * 16
            sbase0 = i * skslot
            for c in range(D):
                plsc.store_scatter(
                    skew, [iota_sk + (sbase0 + c)], tin[c, pl.ds(r0, 16)]
                )
            for k in range(16):
                obase = (r0 + k) * D
                sbase = sbase0 + k * (D + 1)
                for q in range(D // 16):
                    tout[pl.ds(obase + q * 16, 16)] = skew[pl.ds(sbase + q * 16, 16)]

    fire_in(0, tin0, sin0)

    @pl.loop(0, per + 1, step=2)
    def _pair(i2):
        for b in (0, 1):
            t = i2 + b
            tin, tout, sin, sout = bufs[b]
            tin_n, _, sin_n, _ = bufs[1 - b]

            @pl.when(t < cnt)
            def _do():
                @pl.when(t + 1 < cnt)
                def _fire_next():
                    fire_in(t + 1, tin_n, sin_n)

                pltpu.make_async_copy(
                    tt_hbm.at[:, pl.ds(0, RB)], tin, sin
                ).wait()

                @pl.when(t >= 2)
                def _drain_prev_out():
                    pltpu.make_async_copy(
                        tout, scr_hbm.at[pl.ds(0, RB * D)], sout
                    ).wait()

                transpose_buf(tin, tout, skew, RB)
                pltpu.async_copy(
                    tout, scr_hbm.at[pl.ds(col_of(t) * D, RB * D)], sout
                )

    for _, tout, _, sout in bufs:
        pltpu.make_async_copy(tout, scr_hbm.at[pl.ds(0, RB * D)], sout).wait()

    if tail_n:
        @pl.when(wid == nw - 1)
        def _tail():
            pltpu.sync_copy(tt_hbm.at[:, pl.ds(tail0, tail_n)], tin_tail)
            transpose_buf(tin_tail, tout0, skew, tail_n)
            pltpu.sync_copy(
                tout0.at[pl.ds(0, tail_n * D)],
                scr_hbm.at[pl.ds(tail0 * D, tail_n * D)],
            )


def _cbow_body(ids_hbm, table_hbm, out_hbm,
               idx0, idx1, rows0, rows1, ostag, sem0, sem1):
    nc = 2
    ns = 16
    wid = lax.axis_index("c") * ns + lax.axis_index("s")
    b_total = out_hbm.shape[0]
    rows_per_w = b_total // (nc * ns)
    n_chunks = rows_per_w // CB
    row0 = wid * rows_per_w
    inv_l = np.float32(1.0 / L)
    bufs = ((idx0, rows0, sem0), (idx1, rows1, sem1))

    def load_idx_and_fire(g, idxb, rowsb, semb):
        r0 = row0 + g * CB
        pltpu.sync_copy(ids_hbm.at[pl.ds(r0 * L, CB * L)], idxb)
        pltpu.async_copy(table_hbm.at[idxb], rowsb, semb)

    load_idx_and_fire(0, *bufs[0])

    @pl.loop(0, n_chunks, step=2)
    def _chunk2(g2):
        for b in (0, 1):
            g = g2 + b
            idxb, rowsb, semb = bufs[b]

            @pl.when(g + 1 < n_chunks)
            def _fire_next():
                load_idx_and_fire(g + 1, *bufs[1 - b])

            pltpu.make_async_copy(table_hbm.at[idxb], rowsb, semb).wait()

            pos = g % OB
            for cb in range(CB):
                zero = jnp.zeros((16,), jnp.float32)

                @pl.loop(0, L, init_carry=(zero,) * NV, unroll=8)
                def _acc(l, carry):
                    r = cb * L + l
                    return tuple(
                        carry[d] + rowsb[r, pl.ds(d * 16, 16)]
                        for d in range(NV)
                    )

                for d in range(NV):
                    ostag[pos * CB + cb, pl.ds(d * 16, 16)] = _acc[d] * inv_l

            @pl.when(pos == OB - 1)
            def _flush_out():
                pltpu.sync_copy(
                    ostag,
                    out_hbm.at[pl.ds(row0 + (g + 1 - OB) * CB, OB * CB)],
                )


@jax.jit
def kernel(input_ids, table):
    b, l = input_ids.shape
    v = table.shape[0]
    assert l == L and table.shape[1] == D

    mesh = plsc.VectorSubcoreMesh(core_axis_name="c", subcore_axis_name="s")

    k1 = pl.kernel(
        _transpose_body,
        out_type=jax.ShapeDtypeStruct((v * D,), jnp.float32),
        mesh=mesh,
        scratch_types=[
            pltpu.VMEM((D, RB), jnp.float32),
            pltpu.VMEM((D, RB), jnp.float32),
            pltpu.VMEM((RB * D,), jnp.float32),
            pltpu.VMEM((RB * D,), jnp.float32),
            pltpu.VMEM((D, D), jnp.float32),
            pltpu.VMEM(((RB // 16) * 16 * (D + 1),), jnp.float32),
            pltpu.SemaphoreType.DMA,
            pltpu.SemaphoreType.DMA,
            pltpu.SemaphoreType.DMA,
            pltpu.SemaphoreType.DMA,
        ],
        compiler_params=pltpu.CompilerParams(
            needs_layout_passes=False, disable_bounds_checks=True
        ),
    )
    scratch = k1(table.T)

    table2 = scratch.reshape(v, D)
    ids_flat = input_ids.reshape(-1).astype(jnp.int32)

    k2 = pl.kernel(
        _cbow_body,
        out_type=jax.ShapeDtypeStruct((b, D), jnp.float32),
        mesh=mesh,
        scratch_types=[
            pltpu.VMEM((CB * L,), jnp.int32),
            pltpu.VMEM((CB * L,), jnp.int32),
            pltpu.VMEM((CB * L, D), jnp.float32),
            pltpu.VMEM((CB * L, D), jnp.float32),
            pltpu.VMEM((OB * CB, D), jnp.float32),
            pltpu.SemaphoreType.DMA,
            pltpu.SemaphoreType.DMA,
        ],
        compiler_params=pltpu.CompilerParams(use_tc_tiling_on_sc=False),
    )
    return k2(ids_flat, table2)


# speedup vs baseline: 2.6760x; 1.2639x over previous
import jax
import jax.numpy as jnp
import numpy as np
from jax import lax
from jax.experimental import pallas as pl
from jax.experimental.pallas import tpu as pltpu
from jax.experimental.pallas import tpu_sc as plsc

D = 64
L = 200
CB = 4
NV = D // 16


OB = 8
RB = 256


def _transpose_body(tt_hbm, scr_hbm,
                    tin0, tin1, tout0, tout1, tin_tail, skew,
                    sin0, sin1, sout0, sout1):
    nc = 2
    ns = 16
    wid = lax.axis_index("c") * ns + lax.axis_index("s")
    nw = nc * ns
    v = scr_hbm.shape[0] // D
    nblk = v // RB
    tail0 = nblk * RB
    tail_n = v - tail0
    per = nblk // nw
    rem = nblk % nw
    lo = wid * per + jnp.minimum(wid, rem)
    cnt = per + jnp.where(wid < rem, 1, 0)
    iota_sk = lax.iota(jnp.int32, 16) * (D + 1)
    bufs = ((tin0, tout0, sin0, sout0), (tin1, tout1, sin1, sout1))

    def col_of(t):
        return pl.multiple_of((lo + t) * RB, RB)

    def fire_in(t, tin, sin):
        pltpu.async_copy(tt_hbm.at[:, pl.ds(col_of(t), RB)], tin, sin)

    skslot = 16 * (D + 1)

    def transpose_buf(tin, tout, skew, nrows):
        @plsc.parallel_loop(0, nrows // 16, 1, unroll=4)
        def _grp(i):
            r0 = i * 16
            sbase0 = i * skslot
            for c in range(D):
                plsc.store_scatter(
                    skew, [iota_sk + (sbase0 + c)], tin[c, pl.ds(r0, 16)]
                )
            for k in range(16):
                obase = (r0 + k) * D
                sbase = sbase0 + k * (D + 1)
                for q in range(D // 16):
                    tout[pl.ds(obase + q * 16, 16)] = skew[pl.ds(sbase + q * 16, 16)]

    fire_in(0, tin0, sin0)

    @pl.loop(0, per + 1, step=2)
    def _pair(i2):
        for b in (0, 1):
            t = i2 + b
            tin, tout, sin, sout = bufs[b]
            tin_n, _, sin_n, _ = bufs[1 - b]

            @pl.when(t < cnt)
            def _do():
                @pl.when(t + 1 < cnt)
                def _fire_next():
                    fire_in(t + 1, tin_n, sin_n)

                pltpu.make_async_copy(
                    tt_hbm.at[:, pl.ds(0, RB)], tin, sin
                ).wait()

                @pl.when(t >= 2)
                def _drain_prev_out():
                    pltpu.make_async_copy(
                        tout, scr_hbm.at[pl.ds(0, RB * D)], sout
                    ).wait()

                transpose_buf(tin, tout, skew, RB)
                pltpu.async_copy(
                    tout, scr_hbm.at[pl.ds(col_of(t) * D, RB * D)], sout
                )

    for _, tout, _, sout in bufs:
        pltpu.make_async_copy(tout, scr_hbm.at[pl.ds(0, RB * D)], sout).wait()

    if tail_n:
        @pl.when(wid == nw - 1)
        def _tail():
            pltpu.sync_copy(tt_hbm.at[:, pl.ds(tail0, tail_n)], tin_tail)
            transpose_buf(tin_tail, tout0, skew, tail_n)
            pltpu.sync_copy(
                tout0.at[pl.ds(0, tail_n * D)],
                scr_hbm.at[pl.ds(tail0 * D, tail_n * D)],
            )


def _cbow_body(ids_hbm, table_hbm, out_hbm,
               idx0, idx1, rows0, rows1, ostag, sem0, sem1):
    nc = 2
    ns = 16
    wid = lax.axis_index("c") * ns + lax.axis_index("s")
    b_total = out_hbm.shape[0]
    rows_per_w = b_total // (nc * ns)
    n_chunks = rows_per_w // CB
    row0 = wid * rows_per_w
    inv_l = np.float32(1.0 / L)
    bufs = ((idx0, rows0, sem0), (idx1, rows1, sem1))

    def load_idx_and_fire(g, idxb, rowsb, semb):
        r0 = row0 + g * CB
        pltpu.sync_copy(ids_hbm.at[pl.ds(r0 * L, CB * L)], idxb)
        pltpu.async_copy(table_hbm.at[idxb], rowsb, semb)

    load_idx_and_fire(0, *bufs[0])

    @pl.loop(0, n_chunks, step=2)
    def _chunk2(g2):
        for b in (0, 1):
            g = g2 + b
            idxb, rowsb, semb = bufs[b]

            @pl.when(g + 1 < n_chunks)
            def _fire_next():
                load_idx_and_fire(g + 1, *bufs[1 - b])

            pltpu.make_async_copy(table_hbm.at[idxb], rowsb, semb).wait()

            pos = g % OB
            for cb in range(CB):
                zero = jnp.zeros((16,), jnp.float32)

                @pl.loop(0, L, init_carry=(zero,) * NV, unroll=8)
                def _acc(l, carry):
                    r = cb * L + l
                    return tuple(
                        carry[d] + rowsb[r, pl.ds(d * 16, 16)]
                        for d in range(NV)
                    )

                for d in range(NV):
                    ostag[pos * CB + cb, pl.ds(d * 16, 16)] = _acc[d] * inv_l

            @pl.when(pos == OB - 1)
            def _flush_out():
                pltpu.sync_copy(
                    ostag,
                    out_hbm.at[pl.ds(row0 + (g + 1 - OB) * CB, OB * CB)],
                )


@jax.jit
def kernel(input_ids, table):
    b, l = input_ids.shape
    v = table.shape[0]
    assert l == L and table.shape[1] == D

    mesh = plsc.VectorSubcoreMesh(core_axis_name="c", subcore_axis_name="s")

    k1 = pl.kernel(
        _transpose_body,
        out_type=jax.ShapeDtypeStruct((v * D,), jnp.float32),
        mesh=mesh,
        scratch_types=[
            pltpu.VMEM((D, RB), jnp.float32),
            pltpu.VMEM((D, RB), jnp.float32),
            pltpu.VMEM((RB * D,), jnp.float32),
            pltpu.VMEM((RB * D,), jnp.float32),
            pltpu.VMEM((D, D), jnp.float32),
            pltpu.VMEM(((RB // 16) * 16 * (D + 1),), jnp.float32),
            pltpu.SemaphoreType.DMA,
            pltpu.SemaphoreType.DMA,
            pltpu.SemaphoreType.DMA,
            pltpu.SemaphoreType.DMA,
        ],
        compiler_params=pltpu.CompilerParams(
            needs_layout_passes=False, disable_bounds_checks=True
        ),
    )
    scratch = k1(table.T)

    table2 = scratch.reshape(v, D)
    ids_flat = input_ids.reshape(-1).astype(jnp.int32)

    k2 = pl.kernel(
        _cbow_body,
        out_type=jax.ShapeDtypeStruct((b, D), jnp.float32),
        mesh=mesh,
        scratch_types=[
            pltpu.VMEM((CB * L,), jnp.int32),
            pltpu.VMEM((CB * L,), jnp.int32),
            pltpu.VMEM((CB * L, D), jnp.float32),
            pltpu.VMEM((CB * L, D), jnp.float32),
            pltpu.VMEM((OB * CB, D), jnp.float32),
            pltpu.SemaphoreType.DMA,
            pltpu.SemaphoreType.DMA,
        ],
        compiler_params=pltpu.CompilerParams(use_tc_tiling_on_sc=False),
    )
    return k2(ids_flat, table2)
